# baseline (speedup 1.0000x reference)
import functools

import jax
import jax.numpy as jnp
from jax import lax
from jax.experimental import pallas as pl
from jax.experimental.pallas import tpu as pltpu
from jax.experimental.pallas import tpu_sc as plsc

N = 10000
E = 320000
D = 128
DP = D // 2
H = 128
RB = 16

NC = 2
NS = 16
NW = NC * NS
CH = 128
NCHUNK = E // CH
ITERS = (NCHUNK + NW - 1) // NW
GROWS = 80
NGROUP = N // GROWS
GITER = (NGROUP + NS - 1) // NS

_SC_MESH = dict(core_axis_name="c", subcore_axis_name="s",
                num_cores=NC, num_subcores=NS)


def _gather_body(node_hbm, src_hbm, dst_hbm, vi_hbm, vj_hbm,
                 idx_s, idx_d, rows_s, rows_d, sem_s, sem_d):
    cid = lax.axis_index("c")
    sid = lax.axis_index("s")
    wid = sid * NC + cid

    def body(t, carry):
        chunk = t * NW + wid

        @pl.when(chunk < NCHUNK)
        def _():
            base = chunk * CH
            pltpu.sync_copy(src_hbm.at[pl.ds(base, CH)], idx_s)
            pltpu.sync_copy(dst_hbm.at[pl.ds(base, CH)], idx_d)
            cp_s = pltpu.async_copy(node_hbm.at[idx_s], rows_s, sem_s)
            cp_d = pltpu.async_copy(node_hbm.at[idx_d], rows_d, sem_d)
            cp_s.wait()
            cp_d.wait()
            pltpu.sync_copy(rows_s, vi_hbm.at[pl.ds(base, CH)])
            pltpu.sync_copy(rows_d, vj_hbm.at[pl.ds(base, CH)])

        return carry

    lax.fori_loop(0, ITERS, body, 0)


def _sc_gather(node_feat, src, dst):
    f = pl.kernel(
        _gather_body,
        out_type=(jax.ShapeDtypeStruct((E, D), jnp.float32),
                  jax.ShapeDtypeStruct((E, D), jnp.float32)),
        mesh=plsc.VectorSubcoreMesh(**_SC_MESH),
        scratch_types=[
            pltpu.VMEM((CH,), jnp.int32),
            pltpu.VMEM((CH,), jnp.int32),
            pltpu.VMEM((CH, D), jnp.float32),
            pltpu.VMEM((CH, D), jnp.float32),
            pltpu.SemaphoreType.DMA,
            pltpu.SemaphoreType.DMA,
        ],
    )
    return f(node_feat, src, dst)


def _scatter_body(mess_hbm, dst_hbm, node_hbm, out_hbm,
                  acc, idx, rows, sem):
    cid = lax.axis_index("c")
    sid = lax.axis_index("s")
    wid = sid * NC + cid

    def init_body(t, carry):
        g = t * NS + sid

        @pl.when(g < NGROUP)
        def _():
            b = g * GROWS
            pltpu.sync_copy(node_hbm.at[pl.ds(b, GROWS)],
                            acc.at[pl.ds(b, GROWS)])

        return carry

    lax.fori_loop(0, GITER, init_body, 0)
    plsc.subcore_barrier()

    def body(t, carry):
        chunk = t * NW + wid

        @pl.when(chunk < NCHUNK)
        def _():
            base = chunk * CH
            pltpu.sync_copy(dst_hbm.at[pl.ds(base, CH)], idx)
            cp = pltpu.async_copy(mess_hbm.at[pl.ds(base, CH)], rows, sem)
            cp.wait()
            pltpu.sync_copy(rows, acc.at[idx], add=True)

        return carry

    lax.fori_loop(0, ITERS, body, 0)
    plsc.subcore_barrier()

    def dump_body(t, carry):
        g = t * NS + sid

        @pl.when(g < NGROUP)
        def _():
            b = g * GROWS
            pltpu.sync_copy(acc.at[pl.ds(b, GROWS)],
                            out_hbm.at[cid, pl.ds(b, GROWS)])

        return carry

    lax.fori_loop(0, GITER, dump_body, 0)


def _sc_scatter(mess, dst, node_feat):
    f = pl.kernel(
        _scatter_body,
        out_type=jax.ShapeDtypeStruct((NC, N, D), jnp.float32),
        mesh=plsc.VectorSubcoreMesh(**_SC_MESH),
        scratch_types=[
            pltpu.VMEM_SHARED((N, D), jnp.float32),
            pltpu.VMEM((CH,), jnp.int32),
            pltpu.VMEM((CH, D), jnp.float32),
            pltpu.SemaphoreType.DMA,
        ],
    )
    return f(mess, dst, node_feat)


def _silu(x):
    return x * jax.nn.sigmoid(x)


def _mlp_body(vi_ref, vj_ref, ef_ref, rbf_ref,
              w0e_ref, b0e_ref, w1em_ref, b1em_ref, w1eg_ref, b1eg_ref,
              w0n_ref, b0n_ref, w1nm_ref, b1nm_ref, w1ng_ref, b1ng_ref,
              wew_ref, wnw_ref,
              enew_ref, mess_ref):
    ef = ef_ref[...]
    ef_bf = ef.astype(jnp.bfloat16)
    rbf = rbf_ref[...]

    xn = jnp.concatenate([vi_ref[...].astype(jnp.bfloat16),
                          vj_ref[...].astype(jnp.bfloat16)],
                         axis=1)
    w0e = w0e_ref[...]
    z = jnp.dot(xn, w0e[:2 * D], preferred_element_type=jnp.float32)
    z = z + jnp.dot(ef_bf, w0e[2 * D:], preferred_element_type=jnp.float32)
    z = z + b0e_ref[...]
    h = _silu(z[:, :H]).astype(jnp.bfloat16)
    g = _silu(z[:, H:]).astype(jnp.bfloat16)
    h2 = _silu(jnp.dot(h, w1em_ref[...], preferred_element_type=jnp.float32)
               + b1em_ref[...])
    g2 = jax.nn.sigmoid(
        jnp.dot(g, w1eg_ref[...], preferred_element_type=jnp.float32)
        + b1eg_ref[...])
    rew = jnp.dot(rbf, wew_ref[...], preferred_element_type=jnp.float32)
    mij = h2 * g2 * rew
    enew = ef + mij
    enew_ref[...] = enew

    w0n = w0n_ref[...]
    z2 = jnp.dot(xn, w0n[:2 * D], preferred_element_type=jnp.float32)
    z2 = z2 + jnp.dot(enew.astype(jnp.bfloat16), w0n[2 * D:],
                      preferred_element_type=jnp.float32)
    z2 = z2 + b0n_ref[...]
    hn = _silu(z2[:, :H]).astype(jnp.bfloat16)
    gn = _silu(z2[:, H:]).astype(jnp.bfloat16)
    hn2 = _silu(jnp.dot(hn, w1nm_ref[...], preferred_element_type=jnp.float32)
                + b1nm_ref[...])
    gn2 = jax.nn.sigmoid(
        jnp.dot(gn, w1ng_ref[...], preferred_element_type=jnp.float32)
        + b1ng_ref[...])
    rnw = jnp.dot(rbf, wnw_ref[...], preferred_element_type=jnp.float32)
    mess_ref[...] = hn2 * gn2 * rnw


def _tc_mlp(vi, vj, ef, rbf_p, w0e, b0e, w1em, b1em, w1eg, b1eg,
            w0n, b0n, w1nm, b1nm, w1ng, b1ng, wew_p, wnw_p, blk):
    grid = (E // blk,)

    def eb(i):
        return (i, 0)

    def full(arr):
        nd = arr.ndim
        return pl.BlockSpec(arr.shape, lambda i, nd=nd: (0,) * nd)

    in_specs = [
        pl.BlockSpec((blk, D), eb),
        pl.BlockSpec((blk, D), eb),
        pl.BlockSpec((blk, D), eb),
        pl.BlockSpec((blk, RB), eb),
        full(w0e), full(b0e), full(w1em), full(b1em), full(w1eg), full(b1eg),
        full(w0n), full(b0n), full(w1nm), full(b1nm), full(w1ng), full(b1ng),
        full(wew_p), full(wnw_p),
    ]
    out_specs = [pl.BlockSpec((blk, D), eb), pl.BlockSpec((blk, D), eb)]
    return pl.pallas_call(
        _mlp_body,
        grid=grid,
        in_specs=in_specs,
        out_specs=out_specs,
        out_shape=[jax.ShapeDtypeStruct((E, D), jnp.float32),
                   jax.ShapeDtypeStruct((E, D), jnp.float32)],
    )(vi, vj, ef, rbf_p, w0e, b0e, w1em, b1em, w1eg, b1eg,
      w0n, b0n, w1nm, b1nm, w1ng, b1ng, wew_p, wnw_p)


def _combine_body(p_ref, nf_ref, out_ref):
    out_ref[...] = p_ref[0] + p_ref[1] - nf_ref[...]


def _tc_combine(partials, node_feat):
    blk = 2000
    grid = (N // blk,)
    return pl.pallas_call(
        _combine_body,
        grid=grid,
        in_specs=[pl.BlockSpec((NC, blk, D), lambda i: (0, i, 0)),
                  pl.BlockSpec((blk, D), lambda i: (i, 0))],
        out_specs=pl.BlockSpec((blk, D), lambda i: (i, 0)),
        out_shape=jax.ShapeDtypeStruct((N, D), jnp.float32),
    )(partials, node_feat)


def kernel(edge_feat, node_feat, edge_index, rbf, graph_attr,
           W_e_m0, b_e_m0, W_e_m1, b_e_m1, W_e_g0, b_e_g0, W_e_g1, b_e_g1,
           W_n_m0, b_n_m0, W_n_m1, b_n_m1, W_n_g0, b_n_g0, W_n_g1, b_n_g1,
           W_ew, W_nw):
    src = edge_index[0].astype(jnp.int32)
    dst = edge_index[1].astype(jnp.int32)

    vi, vj = _sc_gather(node_feat, src, dst)

    rbf_p = jnp.pad(rbf, ((0, 0), (0, RB - rbf.shape[1])))
    wew_p = jnp.pad(W_ew, ((0, RB - W_ew.shape[0]), (0, 0)))
    wnw_p = jnp.pad(W_nw, ((0, RB - W_nw.shape[0]), (0, 0)))

    w0e = jnp.concatenate([W_e_m0, W_e_g0], axis=1)
    b0e = jnp.concatenate([b_e_m0, b_e_g0])[None, :]
    w0n = jnp.concatenate([W_n_m0, W_n_g0], axis=1)
    b0n = jnp.concatenate([b_n_m0, b_n_g0])[None, :]
    w0e_r = w0e.astype(jnp.bfloat16)
    w0n_r = w0n.astype(jnp.bfloat16)

    edge_new, mess = _tc_mlp(
        vi, vj, edge_feat, rbf_p,
        w0e_r, b0e, W_e_m1.astype(jnp.bfloat16), b_e_m1[None, :],
        W_e_g1.astype(jnp.bfloat16), b_e_g1[None, :],
        w0n_r, b0n, W_n_m1.astype(jnp.bfloat16), b_n_m1[None, :],
        W_n_g1.astype(jnp.bfloat16), b_n_g1[None, :],
        wew_p, wnw_p, 512)

    partials = _sc_scatter(mess, dst, node_feat)
    node_new = _tc_combine(partials, node_feat)
    return (edge_new, node_new, graph_attr)


# speedup vs baseline: 1.1268x; 1.1268x over previous
import functools

import jax
import jax.numpy as jnp
from jax import lax
from jax.experimental import pallas as pl
from jax.experimental.pallas import tpu as pltpu
from jax.experimental.pallas import tpu_sc as plsc

N = 10000
E = 320000
D = 128
DP = D // 2
H = 128
RB = 16

NC = 2
NS = 16
NW = NC * NS
CH = 128
NCHUNK = E // CH
ITERS = (NCHUNK + NW - 1) // NW
GROWS = 80
NGROUP = N // GROWS
GITER = (NGROUP + NS - 1) // NS

_SC_MESH = dict(core_axis_name="c", subcore_axis_name="s",
                num_cores=NC, num_subcores=NS)


def _gather_body(node_hbm, src_hbm, dst_hbm, vi_hbm, vj_hbm,
                 idx_s, idx_d, rows_s, rows_d, sem_s, sem_d):
    cid = lax.axis_index("c")
    sid = lax.axis_index("s")
    wid = sid * NC + cid

    def body(t, carry):
        chunk = t * NW + wid

        @pl.when(chunk < NCHUNK)
        def _():
            base = chunk * CH
            pltpu.sync_copy(src_hbm.at[pl.ds(base, CH)], idx_s)
            pltpu.sync_copy(dst_hbm.at[pl.ds(base, CH)], idx_d)
            cp_s = pltpu.async_copy(node_hbm.at[idx_s], rows_s, sem_s)
            cp_d = pltpu.async_copy(node_hbm.at[idx_d], rows_d, sem_d)
            cp_s.wait()
            cp_d.wait()
            pltpu.sync_copy(rows_s, vi_hbm.at[pl.ds(base, CH)])
            pltpu.sync_copy(rows_d, vj_hbm.at[pl.ds(base, CH)])

        return carry

    lax.fori_loop(0, ITERS, body, 0)


def _sc_gather(node_feat, src, dst):
    f = pl.kernel(
        _gather_body,
        out_type=(jax.ShapeDtypeStruct((E, D), jnp.float32),
                  jax.ShapeDtypeStruct((E, D), jnp.float32)),
        mesh=plsc.VectorSubcoreMesh(**_SC_MESH),
        scratch_types=[
            pltpu.VMEM((CH,), jnp.int32),
            pltpu.VMEM((CH,), jnp.int32),
            pltpu.VMEM((CH, D), jnp.float32),
            pltpu.VMEM((CH, D), jnp.float32),
            pltpu.SemaphoreType.DMA,
            pltpu.SemaphoreType.DMA,
        ],
    )
    return f(node_feat, src, dst)


def _scatter_body(mess_hbm, dst_hbm, node_hbm, out_hbm,
                  acc, idx, rows, sem):
    cid = lax.axis_index("c")
    sid = lax.axis_index("s")
    wid = sid * NC + cid

    def init_body(t, carry):
        g = t * NS + sid

        @pl.when(g < NGROUP)
        def _():
            b = g * GROWS
            pltpu.sync_copy(node_hbm.at[pl.ds(b, GROWS)],
                            acc.at[pl.ds(b, GROWS)])

        return carry

    lax.fori_loop(0, GITER, init_body, 0)
    plsc.subcore_barrier()

    def body(t, carry):
        chunk = t * NW + wid

        @pl.when(chunk < NCHUNK)
        def _():
            base = chunk * CH
            pltpu.sync_copy(dst_hbm.at[pl.ds(base, CH)], idx)
            cp = pltpu.async_copy(mess_hbm.at[pl.ds(base, CH)], rows, sem)
            cp.wait()
            pltpu.sync_copy(rows, acc.at[idx], add=True)

        return carry

    lax.fori_loop(0, ITERS, body, 0)
    plsc.subcore_barrier()

    def dump_body(t, carry):
        g = t * NS + sid

        @pl.when(g < NGROUP)
        def _():
            b = g * GROWS
            pltpu.sync_copy(acc.at[pl.ds(b, GROWS)],
                            out_hbm.at[cid, pl.ds(b, GROWS)])

        return carry

    lax.fori_loop(0, GITER, dump_body, 0)


def _sc_scatter(mess, dst, node_feat):
    f = pl.kernel(
        _scatter_body,
        out_type=jax.ShapeDtypeStruct((NC, N, D), jnp.float32),
        mesh=plsc.VectorSubcoreMesh(**_SC_MESH),
        scratch_types=[
            pltpu.VMEM_SHARED((N, D), jnp.float32),
            pltpu.VMEM((CH,), jnp.int32),
            pltpu.VMEM((CH, D), jnp.float32),
            pltpu.SemaphoreType.DMA,
        ],
    )
    return f(mess, dst, node_feat)


def _silu(x):
    return x * jax.nn.sigmoid(x)


def _mlp_body(vi_ref, vj_ref, ef_ref, rbf_ref,
              w0e_ref, b0e_ref, w1em_ref, b1em_ref, w1eg_ref, b1eg_ref,
              w0n_ref, b0n_ref, w1nm_ref, b1nm_ref, w1ng_ref, b1ng_ref,
              wew_ref, wnw_ref,
              enew_ref, mess_ref):
    ef = ef_ref[...]
    ef_bf = ef.astype(jnp.bfloat16)
    rbf = rbf_ref[...]

    xn = jnp.concatenate([vi_ref[...].astype(jnp.bfloat16),
                          vj_ref[...].astype(jnp.bfloat16)],
                         axis=1)
    w0e = w0e_ref[...]
    z = jnp.dot(xn, w0e[:2 * D], preferred_element_type=jnp.float32)
    z = z + jnp.dot(ef_bf, w0e[2 * D:], preferred_element_type=jnp.float32)
    z = z + b0e_ref[...]
    h = _silu(z[:, :H]).astype(jnp.bfloat16)
    g = _silu(z[:, H:]).astype(jnp.bfloat16)
    h2 = _silu(jnp.dot(h, w1em_ref[...], preferred_element_type=jnp.float32)
               + b1em_ref[...])
    g2 = jax.nn.sigmoid(
        jnp.dot(g, w1eg_ref[...], preferred_element_type=jnp.float32)
        + b1eg_ref[...])
    rew = jnp.dot(rbf, wew_ref[...], preferred_element_type=jnp.float32)
    mij = h2 * g2 * rew
    enew = ef + mij
    enew_ref[...] = enew

    w0n = w0n_ref[...]
    z2 = jnp.dot(xn, w0n[:2 * D], preferred_element_type=jnp.float32)
    z2 = z2 + jnp.dot(enew.astype(jnp.bfloat16), w0n[2 * D:],
                      preferred_element_type=jnp.float32)
    z2 = z2 + b0n_ref[...]
    hn = _silu(z2[:, :H]).astype(jnp.bfloat16)
    gn = _silu(z2[:, H:]).astype(jnp.bfloat16)
    hn2 = _silu(jnp.dot(hn, w1nm_ref[...], preferred_element_type=jnp.float32)
                + b1nm_ref[...])
    gn2 = jax.nn.sigmoid(
        jnp.dot(gn, w1ng_ref[...], preferred_element_type=jnp.float32)
        + b1ng_ref[...])
    rnw = jnp.dot(rbf, wnw_ref[...], preferred_element_type=jnp.float32)
    mess_ref[...] = hn2 * gn2 * rnw


def _tc_mlp(vi, vj, ef, rbf_p, w0e, b0e, w1em, b1em, w1eg, b1eg,
            w0n, b0n, w1nm, b1nm, w1ng, b1ng, wew_p, wnw_p, blk):
    grid = (E // blk,)

    def eb(i):
        return (i, 0)

    def full(arr):
        nd = arr.ndim
        return pl.BlockSpec(arr.shape, lambda i, nd=nd: (0,) * nd)

    in_specs = [
        pl.BlockSpec((blk, D), eb),
        pl.BlockSpec((blk, D), eb),
        pl.BlockSpec((blk, D), eb),
        pl.BlockSpec((blk, RB), eb),
        full(w0e), full(b0e), full(w1em), full(b1em), full(w1eg), full(b1eg),
        full(w0n), full(b0n), full(w1nm), full(b1nm), full(w1ng), full(b1ng),
        full(wew_p), full(wnw_p),
    ]
    out_specs = [pl.BlockSpec((blk, D), eb), pl.BlockSpec((blk, D), eb)]
    return pl.pallas_call(
        _mlp_body,
        grid=grid,
        in_specs=in_specs,
        out_specs=out_specs,
        out_shape=[jax.ShapeDtypeStruct((E, D), jnp.float32),
                   jax.ShapeDtypeStruct((E, D), jnp.float32)],
    )(vi, vj, ef, rbf_p, w0e, b0e, w1em, b1em, w1eg, b1eg,
      w0n, b0n, w1nm, b1nm, w1ng, b1ng, wew_p, wnw_p)


def _combine_body(p_ref, nf_ref, out_ref):
    out_ref[...] = p_ref[0] + p_ref[1] - nf_ref[...]


def _tc_combine(partials, node_feat):
    blk = 2000
    grid = (N // blk,)
    return pl.pallas_call(
        _combine_body,
        grid=grid,
        in_specs=[pl.BlockSpec((NC, blk, D), lambda i: (0, i, 0)),
                  pl.BlockSpec((blk, D), lambda i: (i, 0))],
        out_specs=pl.BlockSpec((blk, D), lambda i: (i, 0)),
        out_shape=jax.ShapeDtypeStruct((N, D), jnp.float32),
    )(partials, node_feat)


def kernel(edge_feat, node_feat, edge_index, rbf, graph_attr,
           W_e_m0, b_e_m0, W_e_m1, b_e_m1, W_e_g0, b_e_g0, W_e_g1, b_e_g1,
           W_n_m0, b_n_m0, W_n_m1, b_n_m1, W_n_g0, b_n_g0, W_n_g1, b_n_g1,
           W_ew, W_nw):
    src = edge_index[0].astype(jnp.int32)
    dst = edge_index[1].astype(jnp.int32)

    vi, vj = _sc_gather(node_feat, src, dst)

    rbf_p = jnp.pad(rbf, ((0, 0), (0, RB - rbf.shape[1])))
    wew_p = jnp.pad(W_ew, ((0, RB - W_ew.shape[0]), (0, 0)))
    wnw_p = jnp.pad(W_nw, ((0, RB - W_nw.shape[0]), (0, 0)))

    w0e = jnp.concatenate([W_e_m0, W_e_g0], axis=1)
    b0e = jnp.concatenate([b_e_m0, b_e_g0])[None, :]
    w0n = jnp.concatenate([W_n_m0, W_n_g0], axis=1)
    b0n = jnp.concatenate([b_n_m0, b_n_g0])[None, :]
    w0e_r = w0e.astype(jnp.bfloat16)
    w0n_r = w0n.astype(jnp.bfloat16)

    edge_new, mess = _tc_mlp(
        vi, vj, edge_feat, rbf_p,
        w0e_r, b0e, W_e_m1.astype(jnp.bfloat16), b_e_m1[None, :],
        W_e_g1.astype(jnp.bfloat16), b_e_g1[None, :],
        w0n_r, b0n, W_n_m1.astype(jnp.bfloat16), b_n_m1[None, :],
        W_n_g1.astype(jnp.bfloat16), b_n_g1[None, :],
        wew_p, wnw_p, 1280)

    partials = _sc_scatter(mess, dst, node_feat)
    node_new = _tc_combine(partials, node_feat)
    return (edge_new, node_new, graph_attr)


# speedup vs baseline: 1.1719x; 1.0400x over previous
import functools

import jax
import jax.numpy as jnp
from jax import lax
from jax.experimental import pallas as pl
from jax.experimental.pallas import tpu as pltpu
from jax.experimental.pallas import tpu_sc as plsc

N = 10000
E = 320000
D = 128
DP = D // 2
H = 128
RB = 16

NC = 2
NS = 16
NW = NC * NS
CH = 128
NCHUNK = E // CH
ITERS = (NCHUNK + NW - 1) // NW
GROWS = 80
NGROUP = N // GROWS
GITER = (NGROUP + NS - 1) // NS

_SC_MESH = dict(core_axis_name="c", subcore_axis_name="s",
                num_cores=NC, num_subcores=NS)


def _gather_body(node_hbm, src_hbm, dst_hbm, vi_hbm, vj_hbm,
                 idx_s, idx_d, rows_s, rows_d, sem_s, sem_d):
    cid = lax.axis_index("c")
    sid = lax.axis_index("s")
    wid = sid * NC + cid

    def body(t, carry):
        chunk = t * NW + wid

        @pl.when(chunk < NCHUNK)
        def _():
            base = chunk * CH
            pltpu.sync_copy(src_hbm.at[pl.ds(base, CH)], idx_s)
            pltpu.sync_copy(dst_hbm.at[pl.ds(base, CH)], idx_d)
            cp_s = pltpu.async_copy(node_hbm.at[idx_s], rows_s, sem_s)
            cp_d = pltpu.async_copy(node_hbm.at[idx_d], rows_d, sem_d)
            cp_s.wait()
            cp_d.wait()
            pltpu.sync_copy(rows_s, vi_hbm.at[pl.ds(base, CH)])
            pltpu.sync_copy(rows_d, vj_hbm.at[pl.ds(base, CH)])

        return carry

    lax.fori_loop(0, ITERS, body, 0)


def _sc_gather(node_feat, src, dst):
    f = pl.kernel(
        _gather_body,
        out_type=(jax.ShapeDtypeStruct((E, D), jnp.float32),
                  jax.ShapeDtypeStruct((E, D), jnp.float32)),
        mesh=plsc.VectorSubcoreMesh(**_SC_MESH),
        scratch_types=[
            pltpu.VMEM((CH,), jnp.int32),
            pltpu.VMEM((CH,), jnp.int32),
            pltpu.VMEM((CH, D), jnp.float32),
            pltpu.VMEM((CH, D), jnp.float32),
            pltpu.SemaphoreType.DMA,
            pltpu.SemaphoreType.DMA,
        ],
    )
    return f(node_feat, src, dst)


def _scatter_body(mess_hbm, dst_hbm, node_hbm, out_hbm,
                  acc, idx, rows, sem):
    cid = lax.axis_index("c")
    sid = lax.axis_index("s")
    wid = sid * NC + cid

    def init_body(t, carry):
        g = t * NS + sid

        @pl.when(g < NGROUP)
        def _():
            b = g * GROWS
            pltpu.sync_copy(node_hbm.at[pl.ds(b, GROWS)],
                            acc.at[pl.ds(b, GROWS)])

        return carry

    lax.fori_loop(0, GITER, init_body, 0)
    plsc.subcore_barrier()

    def body(t, carry):
        chunk = t * NW + wid

        @pl.when(chunk < NCHUNK)
        def _():
            base = chunk * CH
            pltpu.sync_copy(dst_hbm.at[pl.ds(base, CH)], idx)
            cp = pltpu.async_copy(mess_hbm.at[pl.ds(base, CH)], rows, sem)
            cp.wait()
            pltpu.sync_copy(rows, acc.at[idx], add=True)

        return carry

    lax.fori_loop(0, ITERS, body, 0)
    plsc.subcore_barrier()

    def dump_body(t, carry):
        g = t * NS + sid

        @pl.when(g < NGROUP)
        def _():
            b = g * GROWS
            pltpu.sync_copy(acc.at[pl.ds(b, GROWS)],
                            out_hbm.at[cid, pl.ds(b, GROWS)])

        return carry

    lax.fori_loop(0, GITER, dump_body, 0)


def _sc_scatter(mess, dst, node_feat):
    f = pl.kernel(
        _scatter_body,
        out_type=jax.ShapeDtypeStruct((NC, N, D), jnp.float32),
        mesh=plsc.VectorSubcoreMesh(**_SC_MESH),
        scratch_types=[
            pltpu.VMEM_SHARED((N, D), jnp.float32),
            pltpu.VMEM((CH,), jnp.int32),
            pltpu.VMEM((CH, D), jnp.float32),
            pltpu.SemaphoreType.DMA,
        ],
    )
    return f(mess, dst, node_feat)


def _silu(x):
    return x * jax.nn.sigmoid(x)


def _mlp_body(vi_ref, vj_ref, ef_ref, rbf_ref,
              w0e_ref, b0e_ref, w1em_ref, b1em_ref, w1eg_ref, b1eg_ref,
              w0n_ref, b0n_ref, w1nm_ref, b1nm_ref, w1ng_ref, b1ng_ref,
              wew_ref, wnw_ref,
              enew_ref, mess_ref):
    ef = ef_ref[...]
    ef_bf = ef.astype(jnp.bfloat16)
    rbf = rbf_ref[...]

    xn = jnp.concatenate([vi_ref[...].astype(jnp.bfloat16),
                          vj_ref[...].astype(jnp.bfloat16)],
                         axis=1)
    w0e = w0e_ref[...]
    z = jnp.dot(xn, w0e[:2 * D], preferred_element_type=jnp.float32)
    z = z + jnp.dot(ef_bf, w0e[2 * D:], preferred_element_type=jnp.float32)
    z = z + b0e_ref[...]
    h = _silu(z[:, :H]).astype(jnp.bfloat16)
    g = _silu(z[:, H:]).astype(jnp.bfloat16)
    h2 = _silu(jnp.dot(h, w1em_ref[...], preferred_element_type=jnp.float32)
               + b1em_ref[...])
    g2 = jax.nn.sigmoid(
        jnp.dot(g, w1eg_ref[...], preferred_element_type=jnp.float32)
        + b1eg_ref[...])
    rew = jnp.dot(rbf, wew_ref[...], preferred_element_type=jnp.float32)
    mij = h2 * g2 * rew
    enew = ef + mij
    enew_ref[...] = enew

    w0n = w0n_ref[...]
    z2 = jnp.dot(xn, w0n[:2 * D], preferred_element_type=jnp.float32)
    z2 = z2 + jnp.dot(enew.astype(jnp.bfloat16), w0n[2 * D:],
                      preferred_element_type=jnp.float32)
    z2 = z2 + b0n_ref[...]
    hn = _silu(z2[:, :H]).astype(jnp.bfloat16)
    gn = _silu(z2[:, H:]).astype(jnp.bfloat16)
    hn2 = _silu(jnp.dot(hn, w1nm_ref[...], preferred_element_type=jnp.float32)
                + b1nm_ref[...])
    gn2 = jax.nn.sigmoid(
        jnp.dot(gn, w1ng_ref[...], preferred_element_type=jnp.float32)
        + b1ng_ref[...])
    rnw = jnp.dot(rbf, wnw_ref[...], preferred_element_type=jnp.float32)
    mess_ref[...] = hn2 * gn2 * rnw


def _tc_mlp(vi, vj, ef, rbf_p, w0e, b0e, w1em, b1em, w1eg, b1eg,
            w0n, b0n, w1nm, b1nm, w1ng, b1ng, wew_p, wnw_p, blk):
    grid = (E // blk,)

    def eb(i):
        return (i, 0)

    def full(arr):
        nd = arr.ndim
        return pl.BlockSpec(arr.shape, lambda i, nd=nd: (0,) * nd)

    in_specs = [
        pl.BlockSpec((blk, D), eb),
        pl.BlockSpec((blk, D), eb),
        pl.BlockSpec((blk, D), eb),
        pl.BlockSpec((blk, RB), eb),
        full(w0e), full(b0e), full(w1em), full(b1em), full(w1eg), full(b1eg),
        full(w0n), full(b0n), full(w1nm), full(b1nm), full(w1ng), full(b1ng),
        full(wew_p), full(wnw_p),
    ]
    out_specs = [pl.BlockSpec((blk, D), eb), pl.BlockSpec((blk, D), eb)]
    return pl.pallas_call(
        _mlp_body,
        grid=grid,
        in_specs=in_specs,
        out_specs=out_specs,
        out_shape=[jax.ShapeDtypeStruct((E, D), jnp.float32),
                   jax.ShapeDtypeStruct((E, D), jnp.float32)],
    )(vi, vj, ef, rbf_p, w0e, b0e, w1em, b1em, w1eg, b1eg,
      w0n, b0n, w1nm, b1nm, w1ng, b1ng, wew_p, wnw_p)


def _combine_body(p_ref, nf_ref, out_ref):
    out_ref[...] = p_ref[0] + p_ref[1] - nf_ref[...]


def _tc_combine(partials, node_feat):
    blk = 2000
    grid = (N // blk,)
    return pl.pallas_call(
        _combine_body,
        grid=grid,
        in_specs=[pl.BlockSpec((NC, blk, D), lambda i: (0, i, 0)),
                  pl.BlockSpec((blk, D), lambda i: (i, 0))],
        out_specs=pl.BlockSpec((blk, D), lambda i: (i, 0)),
        out_shape=jax.ShapeDtypeStruct((N, D), jnp.float32),
    )(partials, node_feat)


def kernel(edge_feat, node_feat, edge_index, rbf, graph_attr,
           W_e_m0, b_e_m0, W_e_m1, b_e_m1, W_e_g0, b_e_g0, W_e_g1, b_e_g1,
           W_n_m0, b_n_m0, W_n_m1, b_n_m1, W_n_g0, b_n_g0, W_n_g1, b_n_g1,
           W_ew, W_nw):
    src = edge_index[0].astype(jnp.int32)
    dst = edge_index[1].astype(jnp.int32)

    vi, vj = _sc_gather(node_feat, src, dst)

    rbf_p = jnp.pad(rbf, ((0, 0), (0, RB - rbf.shape[1])))
    wew_p = jnp.pad(W_ew, ((0, RB - W_ew.shape[0]), (0, 0)))
    wnw_p = jnp.pad(W_nw, ((0, RB - W_nw.shape[0]), (0, 0)))

    w0e = jnp.concatenate([W_e_m0, W_e_g0], axis=1)
    b0e = jnp.concatenate([b_e_m0, b_e_g0])[None, :]
    w0n = jnp.concatenate([W_n_m0, W_n_g0], axis=1)
    b0n = jnp.concatenate([b_n_m0, b_n_g0])[None, :]
    w0e_r = w0e.astype(jnp.bfloat16)
    w0n_r = w0n.astype(jnp.bfloat16)

    edge_new, mess = _tc_mlp(
        vi, vj, edge_feat, rbf_p,
        w0e_r, b0e, W_e_m1.astype(jnp.bfloat16), b_e_m1[None, :],
        W_e_g1.astype(jnp.bfloat16), b_e_g1[None, :],
        w0n_r, b0n, W_n_m1.astype(jnp.bfloat16), b_n_m1[None, :],
        W_n_g1.astype(jnp.bfloat16), b_n_g1[None, :],
        wew_p, wnw_p, 2560)

    partials = _sc_scatter(mess, dst, node_feat)
    node_new = _tc_combine(partials, node_feat)
    return (edge_new, node_new, graph_attr)


# speedup vs baseline: 1.3933x; 1.1889x over previous
import functools

import jax
import jax.numpy as jnp
from jax import lax
from jax.experimental import pallas as pl
from jax.experimental.pallas import tpu as pltpu
from jax.experimental.pallas import tpu_sc as plsc

N = 10000
E = 320000
D = 128
DP = D // 2
H = 128
RB = 16

NC = 2
NS = 16
NW = NC * NS
CH = 128
NCHUNK = E // CH
ITERS = (NCHUNK + NW - 1) // NW

NSLAB = 4
ES = E // NSLAB
SCHUNK = ES // CH
SITERS = (SCHUNK + NW - 1) // NW
BLK = 2000
BPS = ES // BLK
GROWS = 80
NGROUP = N // GROWS
GITER = (NGROUP + NS - 1) // NS

_SC_MESH = dict(core_axis_name="c", subcore_axis_name="s",
                num_cores=NC, num_subcores=NS)


def _gather_body(node_hbm, src_hbm, dst_hbm, vi_hbm, vj_hbm,
                 idx_s, idx_d, rows_s, rows_d, sem_s, sem_d):
    cid = lax.axis_index("c")
    sid = lax.axis_index("s")
    wid = sid * NC + cid

    def body(t, carry):
        chunk = t * NW + wid

        @pl.when(chunk < SCHUNK)
        def _():
            base = chunk * CH
            pltpu.sync_copy(src_hbm.at[pl.ds(base, CH)], idx_s)
            pltpu.sync_copy(dst_hbm.at[pl.ds(base, CH)], idx_d)
            cp_s = pltpu.async_copy(node_hbm.at[idx_s], rows_s, sem_s)
            cp_d = pltpu.async_copy(node_hbm.at[idx_d], rows_d, sem_d)
            cp_s.wait()
            cp_d.wait()
            pltpu.sync_copy(rows_s, vi_hbm.at[pl.ds(base, CH)])
            pltpu.sync_copy(rows_d, vj_hbm.at[pl.ds(base, CH)])

        return carry

    lax.fori_loop(0, SITERS, body, 0)


def _sc_gather(node_feat, src_slab, dst_slab):
    f = pl.kernel(
        _gather_body,
        out_type=(jax.ShapeDtypeStruct((ES, D), jnp.float32),
                  jax.ShapeDtypeStruct((ES, D), jnp.float32)),
        mesh=plsc.VectorSubcoreMesh(**_SC_MESH),
        scratch_types=[
            pltpu.VMEM((CH,), jnp.int32),
            pltpu.VMEM((CH,), jnp.int32),
            pltpu.VMEM((CH, D), jnp.float32),
            pltpu.VMEM((CH, D), jnp.float32),
            pltpu.SemaphoreType.DMA,
            pltpu.SemaphoreType.DMA,
        ],
    )
    return f(node_feat, src_slab, dst_slab)


def _scatter_body(mess_hbm, dst_hbm, node_hbm, out_hbm,
                  acc, idx, rows, sem):
    cid = lax.axis_index("c")
    sid = lax.axis_index("s")
    wid = sid * NC + cid

    def init_body(t, carry):
        g = t * NS + sid

        @pl.when(g < NGROUP)
        def _():
            b = g * GROWS
            pltpu.sync_copy(node_hbm.at[pl.ds(b, GROWS)],
                            acc.at[pl.ds(b, GROWS)])

        return carry

    lax.fori_loop(0, GITER, init_body, 0)
    plsc.subcore_barrier()

    def body(t, carry):
        chunk = t * NW + wid

        @pl.when(chunk < NCHUNK)
        def _():
            base = chunk * CH
            pltpu.sync_copy(dst_hbm.at[pl.ds(base, CH)], idx)
            cp = pltpu.async_copy(mess_hbm.at[pl.ds(base, CH)], rows, sem)
            cp.wait()
            pltpu.sync_copy(rows, acc.at[idx], add=True)

        return carry

    lax.fori_loop(0, ITERS, body, 0)
    plsc.subcore_barrier()

    def dump_body(t, carry):
        g = t * NS + sid

        @pl.when(g < NGROUP)
        def _():
            b = g * GROWS
            pltpu.sync_copy(acc.at[pl.ds(b, GROWS)],
                            out_hbm.at[cid, pl.ds(b, GROWS)])

        return carry

    lax.fori_loop(0, GITER, dump_body, 0)


def _sc_scatter(mess, dst, node_feat):
    f = pl.kernel(
        _scatter_body,
        out_type=jax.ShapeDtypeStruct((NC, N, D), jnp.float32),
        mesh=plsc.VectorSubcoreMesh(**_SC_MESH),
        scratch_types=[
            pltpu.VMEM_SHARED((N, D), jnp.float32),
            pltpu.VMEM((CH,), jnp.int32),
            pltpu.VMEM((CH, D), jnp.float32),
            pltpu.SemaphoreType.DMA,
        ],
    )
    return f(mess, dst, node_feat)


def _silu(x):
    return x * jax.nn.sigmoid(x)


def _mlp_body(vi_ref, vj_ref, ef_ref, rbf_ref,
              w0e_ref, b0e_ref, w1em_ref, b1em_ref, w1eg_ref, b1eg_ref,
              w0n_ref, b0n_ref, w1nm_ref, b1nm_ref, w1ng_ref, b1ng_ref,
              wew_ref, wnw_ref,
              enew_ref, mess_ref):
    ef = ef_ref[...]
    ef_bf = ef.astype(jnp.bfloat16)
    rbf = rbf_ref[...]

    xn = jnp.concatenate([vi_ref[...].astype(jnp.bfloat16),
                          vj_ref[...].astype(jnp.bfloat16)],
                         axis=1)
    w0e = w0e_ref[...]
    z = jnp.dot(xn, w0e[:2 * D], preferred_element_type=jnp.float32)
    z = z + jnp.dot(ef_bf, w0e[2 * D:], preferred_element_type=jnp.float32)
    z = z + b0e_ref[...]
    h = _silu(z[:, :H]).astype(jnp.bfloat16)
    g = _silu(z[:, H:]).astype(jnp.bfloat16)
    h2 = _silu(jnp.dot(h, w1em_ref[...], preferred_element_type=jnp.float32)
               + b1em_ref[...])
    g2 = jax.nn.sigmoid(
        jnp.dot(g, w1eg_ref[...], preferred_element_type=jnp.float32)
        + b1eg_ref[...])
    rew = jnp.dot(rbf, wew_ref[...], preferred_element_type=jnp.float32)
    mij = h2 * g2 * rew
    enew = ef + mij
    enew_ref[...] = enew

    w0n = w0n_ref[...]
    z2 = jnp.dot(xn, w0n[:2 * D], preferred_element_type=jnp.float32)
    z2 = z2 + jnp.dot(enew.astype(jnp.bfloat16), w0n[2 * D:],
                      preferred_element_type=jnp.float32)
    z2 = z2 + b0n_ref[...]
    hn = _silu(z2[:, :H]).astype(jnp.bfloat16)
    gn = _silu(z2[:, H:]).astype(jnp.bfloat16)
    hn2 = _silu(jnp.dot(hn, w1nm_ref[...], preferred_element_type=jnp.float32)
                + b1nm_ref[...])
    gn2 = jax.nn.sigmoid(
        jnp.dot(gn, w1ng_ref[...], preferred_element_type=jnp.float32)
        + b1ng_ref[...])
    rnw = jnp.dot(rbf, wnw_ref[...], preferred_element_type=jnp.float32)
    mess_ref[...] = hn2 * gn2 * rnw


def _mlp_body_alias(vi_ref, vj_ref, ef_ref, rbf_ref,
                    w0e_ref, b0e_ref, w1em_ref, b1em_ref, w1eg_ref, b1eg_ref,
                    w0n_ref, b0n_ref, w1nm_ref, b1nm_ref, w1ng_ref, b1ng_ref,
                    wew_ref, wnw_ref, enew_in, mess_in,
                    enew_ref, mess_ref):
    del enew_in, mess_in
    _mlp_body(vi_ref, vj_ref, ef_ref, rbf_ref,
              w0e_ref, b0e_ref, w1em_ref, b1em_ref, w1eg_ref, b1eg_ref,
              w0n_ref, b0n_ref, w1nm_ref, b1nm_ref, w1ng_ref, b1ng_ref,
              wew_ref, wnw_ref, enew_ref, mess_ref)


def _tc_mlp_slab(k, vi, vj, ef, rbf_p, weights, enew_acc, mess_acc):
    off = k * BPS

    def sb(i, off=off):
        return (off + i, 0)

    def full(arr):
        nd = arr.ndim
        return pl.BlockSpec(arr.shape, lambda i, nd=nd: (0,) * nd)

    in_specs = [
        pl.BlockSpec((BLK, D), lambda i: (i, 0)),
        pl.BlockSpec((BLK, D), lambda i: (i, 0)),
        pl.BlockSpec((BLK, D), sb),
        pl.BlockSpec((BLK, RB), sb),
    ] + [full(w) for w in weights]
    args = [vi, vj, ef, rbf_p, *weights]
    if k == 0:
        body = _mlp_body
        aliases = {}
    else:
        body = _mlp_body_alias
        in_specs += [pl.BlockSpec(memory_space=pl.ANY),
                     pl.BlockSpec(memory_space=pl.ANY)]
        args += [enew_acc, mess_acc]
        aliases = {18: 0, 19: 1}
    out_specs = [pl.BlockSpec((BLK, D), sb), pl.BlockSpec((BLK, D), sb)]
    return pl.pallas_call(
        body,
        grid=(BPS,),
        in_specs=in_specs,
        out_specs=out_specs,
        out_shape=[jax.ShapeDtypeStruct((E, D), jnp.float32),
                   jax.ShapeDtypeStruct((E, D), jnp.float32)],
        input_output_aliases=aliases,
    )(*args)


def _combine_body(p_ref, nf_ref, out_ref):
    out_ref[...] = p_ref[0] + p_ref[1] - nf_ref[...]


def _tc_combine(partials, node_feat):
    blk = 2000
    grid = (N // blk,)
    return pl.pallas_call(
        _combine_body,
        grid=grid,
        in_specs=[pl.BlockSpec((NC, blk, D), lambda i: (0, i, 0)),
                  pl.BlockSpec((blk, D), lambda i: (i, 0))],
        out_specs=pl.BlockSpec((blk, D), lambda i: (i, 0)),
        out_shape=jax.ShapeDtypeStruct((N, D), jnp.float32),
    )(partials, node_feat)


def kernel(edge_feat, node_feat, edge_index, rbf, graph_attr,
           W_e_m0, b_e_m0, W_e_m1, b_e_m1, W_e_g0, b_e_g0, W_e_g1, b_e_g1,
           W_n_m0, b_n_m0, W_n_m1, b_n_m1, W_n_g0, b_n_g0, W_n_g1, b_n_g1,
           W_ew, W_nw):
    src = edge_index[0].astype(jnp.int32)
    dst = edge_index[1].astype(jnp.int32)

    rbf_p = jnp.pad(rbf, ((0, 0), (0, RB - rbf.shape[1])))
    wew_p = jnp.pad(W_ew, ((0, RB - W_ew.shape[0]), (0, 0)))
    wnw_p = jnp.pad(W_nw, ((0, RB - W_nw.shape[0]), (0, 0)))

    w0e = jnp.concatenate([W_e_m0, W_e_g0], axis=1)
    b0e = jnp.concatenate([b_e_m0, b_e_g0])[None, :]
    w0n = jnp.concatenate([W_n_m0, W_n_g0], axis=1)
    b0n = jnp.concatenate([b_n_m0, b_n_g0])[None, :]
    weights = [
        w0e.astype(jnp.bfloat16), b0e,
        W_e_m1.astype(jnp.bfloat16), b_e_m1[None, :],
        W_e_g1.astype(jnp.bfloat16), b_e_g1[None, :],
        w0n.astype(jnp.bfloat16), b0n,
        W_n_m1.astype(jnp.bfloat16), b_n_m1[None, :],
        W_n_g1.astype(jnp.bfloat16), b_n_g1[None, :],
        wew_p, wnw_p,
    ]

    edge_new = mess = None
    for k in range(NSLAB):
        sl = slice(k * ES, (k + 1) * ES)
        vi_k, vj_k = _sc_gather(node_feat, src[sl], dst[sl])
        edge_new, mess = _tc_mlp_slab(k, vi_k, vj_k, edge_feat, rbf_p,
                                      weights, edge_new, mess)

    partials = _sc_scatter(mess, dst, node_feat)
    node_new = _tc_combine(partials, node_feat)
    return (edge_new, node_new, graph_attr)
